# baseline (speedup 1.0000x reference)
import functools
import math

import jax
import jax.numpy as jnp
from jax import lax
from jax.experimental import pallas as pl
from jax.experimental.pallas import tpu as pltpu

H = 128
XP = 16
EB = 2000
NB = 1000


def _silu(v):
    return v * jax.nn.sigmoid(v)


def _edge_tc(ga, gb, rel, lp, need_coord):
    E = ga.shape[0]
    wd = lp["edge0"]["w"][2 * H].reshape(1, H)
    b0 = lp["edge0"]["b"].reshape(1, H)
    w1 = lp["edge1"]["w"]
    b1 = lp["edge1"]["b"].reshape(1, H)
    wa = jnp.zeros((H, 8), jnp.float32).at[:, 0].set(lp["att"]["w"][:, 0])
    ba = jnp.zeros((1, 8), jnp.float32).at[0, 0].set(lp["att"]["b"][0])
    wc0 = lp["coord0"]["w"]
    bc0 = lp["coord0"]["b"].reshape(1, H)
    wc1 = jnp.zeros((H, 8), jnp.float32).at[:, 0].set(lp["coord1"]["w"][:, 0])

    def body(ga_ref, gb_ref, rel_ref, wd_ref, b0_ref, w1_ref, b1_ref, wa_ref,
             ba_ref, wc0_ref, bc0_ref, wc1_ref, sm_ref, sx_ref):
        rel_v = rel_ref[...]
        dist = jnp.sqrt(jnp.sum(rel_v * rel_v, axis=1, keepdims=True))
        z = ga_ref[...] + gb_ref[...] + dist * wd_ref[...] + b0_ref[...]
        u = _silu(z)
        m = _silu(jnp.dot(u, w1_ref[...],
                          preferred_element_type=jnp.float32) + b1_ref[...])
        a8 = jnp.dot(m, wa_ref[...], preferred_element_type=jnp.float32)
        att = jax.nn.sigmoid(a8[:, :1] + ba_ref[...][:, :1])
        sm_ref[...] = att * m
        if need_coord:
            tcw = _silu(jnp.dot(m, wc0_ref[...],
                                preferred_element_type=jnp.float32)
                        + bc0_ref[...])
            c8 = jnp.dot(tcw, wc1_ref[...],
                         preferred_element_type=jnp.float32)
            sx_ref[...] = c8[:, :1] * (rel_v / (dist + 1e-8))
        else:
            sx_ref[...] = jnp.zeros_like(rel_v)

    grid = (E // EB,)
    row_spec = lambda w: pl.BlockSpec((EB, w), lambda i: (i, 0))
    full = lambda a: pl.BlockSpec(a.shape, lambda i: (0,) * a.ndim)
    sm, sx = pl.pallas_call(
        body,
        grid=grid,
        in_specs=[row_spec(H), row_spec(H), row_spec(XP), full(wd), full(b0),
                  full(w1), full(b1), full(wa), full(ba), full(wc0),
                  full(bc0), full(wc1)],
        out_specs=[row_spec(H), row_spec(XP)],
        out_shape=[jax.ShapeDtypeStruct((E, H), jnp.float32),
                   jax.ShapeDtypeStruct((E, XP), jnp.float32)],
    )(ga, gb, rel, wd, b0, w1, b1, wa, ba, wc0, bc0, wc1)
    return sm, sx


def _node_tc(h, mi0, mi1, dx0, dx1, x, lp, nxt):
    N = h.shape[0]
    wn0 = lp["node0"]["w"]
    wn0a, wn0b = wn0[:H], wn0[H:]
    bn0 = lp["node0"]["b"].reshape(1, H)
    wn1 = lp["node1"]["w"]
    bn1 = lp["node1"]["b"].reshape(1, H)
    w0 = nxt["edge0"]["w"]
    w0a, w0b = w0[:H], w0[H:2 * H]

    def body(h_ref, mi0_ref, mi1_ref, dx0_ref, dx1_ref, x_ref, wn0a_ref,
             wn0b_ref, bn0_ref, wn1_ref, bn1_ref, w0a_ref, w0b_ref,
             h_out, x_out, ha_out, hb_out):
        h_v = h_ref[...]
        mi = mi0_ref[...] + mi1_ref[...]
        u = _silu(jnp.dot(h_v, wn0a_ref[...],
                          preferred_element_type=jnp.float32)
                  + jnp.dot(mi, wn0b_ref[...],
                            preferred_element_type=jnp.float32)
                  + bn0_ref[...])
        hn = jnp.dot(u, wn1_ref[...],
                     preferred_element_type=jnp.float32) + bn1_ref[...]
        hv = h_v + hn
        h_out[...] = hv
        x_out[...] = x_ref[...] + dx0_ref[...] + dx1_ref[...]
        ha_out[...] = jnp.dot(hv, w0a_ref[...],
                              preferred_element_type=jnp.float32)
        hb_out[...] = jnp.dot(hv, w0b_ref[...],
                              preferred_element_type=jnp.float32)

    grid = (N // NB,)
    row_spec = lambda w: pl.BlockSpec((NB, w), lambda i: (i, 0))
    full = lambda a: pl.BlockSpec(a.shape, lambda i: (0,) * a.ndim)
    h2, x2, ha, hb = pl.pallas_call(
        body,
        grid=grid,
        in_specs=[row_spec(H), row_spec(H), row_spec(H), row_spec(XP),
                  row_spec(XP), row_spec(XP)] + [full(a) for a in
                  (wn0a, wn0b, bn0, wn1, bn1, w0a, w0b)],
        out_specs=[row_spec(H), row_spec(XP), row_spec(H), row_spec(H)],
        out_shape=[jax.ShapeDtypeStruct((N, H), jnp.float32),
                   jax.ShapeDtypeStruct((N, XP), jnp.float32),
                   jax.ShapeDtypeStruct((N, H), jnp.float32),
                   jax.ShapeDtypeStruct((N, H), jnp.float32)],
    )(h, mi0, mi1, dx0, dx1, x, wn0a, wn0b, bn0, wn1, bn1, w0a, w0b)
    return h2, x2, ha, hb


def _prologue_tc(onehot, table, tc_add, first):
    N = onehot.shape[0]
    w0 = first["edge0"]["w"]
    w0a, w0b = w0[:H], w0[H:2 * H]

    def body(oh_ref, tab_ref, add_ref, w0a_ref, w0b_ref,
             h_out, ha_out, hb_out):
        hv = jnp.dot(oh_ref[...], tab_ref[...],
                     preferred_element_type=jnp.float32) + add_ref[...]
        h_out[...] = hv
        ha_out[...] = jnp.dot(hv, w0a_ref[...],
                              preferred_element_type=jnp.float32)
        hb_out[...] = jnp.dot(hv, w0b_ref[...],
                              preferred_element_type=jnp.float32)

    grid = (N // NB,)
    row_spec = lambda w: pl.BlockSpec((NB, w), lambda i: (i, 0))
    full = lambda a: pl.BlockSpec(a.shape, lambda i: (0,) * a.ndim)
    h0, ha, hb = pl.pallas_call(
        body,
        grid=grid,
        in_specs=[row_spec(H), full(table), full(tc_add), full(w0a),
                  full(w0b)],
        out_specs=[row_spec(H), row_spec(H), row_spec(H)],
        out_shape=[jax.ShapeDtypeStruct((N, H), jnp.float32)] * 3,
    )(onehot, table, tc_add, w0a, w0b)
    return h0, ha, hb


def _final_tc(h, mi0, mi1, lp, norm, whead, bhead):
    N = h.shape[0]
    wn0 = lp["node0"]["w"]
    wn0a, wn0b = wn0[:H], wn0[H:]
    bn0 = lp["node0"]["b"].reshape(1, H)
    wn1 = lp["node1"]["w"]
    bn1 = lp["node1"]["b"].reshape(1, H)
    g = norm["g"].reshape(1, H)
    b = norm["b"].reshape(1, H)

    def body(h_ref, mi0_ref, mi1_ref, wn0a_ref, wn0b_ref, bn0_ref, wn1_ref,
             bn1_ref, g_ref, b_ref, wh_ref, bh_ref, out_ref):
        h_v = h_ref[...]
        mi = mi0_ref[...] + mi1_ref[...]
        u = _silu(jnp.dot(h_v, wn0a_ref[...],
                          preferred_element_type=jnp.float32)
                  + jnp.dot(mi, wn0b_ref[...],
                            preferred_element_type=jnp.float32)
                  + bn0_ref[...])
        hv = h_v + jnp.dot(u, wn1_ref[...],
                           preferred_element_type=jnp.float32) + bn1_ref[...]
        mu = jnp.mean(hv, axis=1, keepdims=True)
        var = jnp.mean((hv - mu) ** 2, axis=1, keepdims=True)
        hn = (hv - mu) / jnp.sqrt(var + 1e-5) * g_ref[...] + b_ref[...]
        out_ref[...] = jnp.dot(hn, wh_ref[...],
                               preferred_element_type=jnp.float32) + bh_ref[...]

    grid = (N // NB,)
    row_spec = lambda w: pl.BlockSpec((NB, w), lambda i: (i, 0))
    full = lambda a: pl.BlockSpec(a.shape, lambda i: (0,) * a.ndim)
    out = pl.pallas_call(
        body,
        grid=grid,
        in_specs=[row_spec(H), row_spec(H), row_spec(H)] + [full(a) for a in
                  (wn0a, wn0b, bn0, wn1, bn1, g, b, whead, bhead)],
        out_specs=row_spec(H),
        out_shape=jax.ShapeDtypeStruct((N, H), jnp.float32),
    )(h, mi0, mi1, wn0a, wn0b, bn0, wn1, bn1, g, b, whead, bhead)
    return out


def _gather(ha, hb, x, row, col):
    ga = ha[row]
    gb = hb[col]
    rel = x[row] - x[col]
    return ga, gb, rel


def _scatter(sm, sx, row, n, need_coord):
    mi = jnp.zeros((n, H), jnp.float32).at[row].add(sm)
    z16 = jnp.zeros((n, XP), jnp.float32)
    dx = z16.at[row].add(sx) if need_coord else z16
    return mi, jnp.zeros((n, H), jnp.float32), dx, z16


def _apply(p, v):
    y = v @ p["w"]
    if "b" in p:
        y = y + p["b"]
    return y


def kernel(atom_types, coords, edge_index, t, anchor_features,
           distance_constraints, coordination_constraints, params):
    N = coords.shape[0]

    half = H // 2
    freqs = jnp.exp(jnp.arange(half, dtype=jnp.float32)
                    * (-(math.log(10000.0) / (half - 1))))
    te = t.astype(jnp.float32)[:, None] * freqs[None, :]
    te = jnp.concatenate([jnp.sin(te), jnp.cos(te)], axis=-1)
    te = _apply(params["time_mlp"][1],
                jax.nn.silu(_apply(params["time_mlp"][0], te)))
    a = _apply(params["cond_anchor"][1], jax.nn.silu(
        _apply(params["cond_anchor"][0], anchor_features))).mean(
            axis=0, keepdims=True)
    d = _apply(params["cond_dist"][1], jax.nn.silu(
        _apply(params["cond_dist"][0], distance_constraints))).mean(
            axis=0, keepdims=True)
    c = _apply(params["cond_coord"][1], jax.nn.silu(
        _apply(params["cond_coord"][0], coordination_constraints))).mean(
            axis=0, keepdims=True)
    comb = jnp.concatenate([a, d, c], axis=-1)
    fz = _apply(params["fusion_lin0"], comb)
    mu = fz.mean(axis=-1, keepdims=True)
    var = ((fz - mu) ** 2).mean(axis=-1, keepdims=True)
    fz = (fz - mu) / jnp.sqrt(var + 1e-5) * params["fusion_ln"]["g"] \
        + params["fusion_ln"]["b"]
    ce = _apply(params["fusion_lin1"], jax.nn.silu(fz))
    tc_add = _apply(params["time_proj"], te) + _apply(params["cond_proj"], ce)

    x = jnp.zeros((N, XP), jnp.float32).at[:, :3].set(coords)
    onehot = jax.nn.one_hot(atom_types, H, dtype=jnp.float32)
    table = jnp.zeros((H, H), jnp.float32).at[
        :params["atom_embed"].shape[0]].set(params["atom_embed"])
    nat = params["atom_head"]["w"].shape[1]
    whead = jnp.zeros((H, H), jnp.float32)
    whead = whead.at[:, :nat].set(params["atom_head"]["w"])
    whead = whead.at[:, nat:nat + 3].set(params["coord_head"]["w"])
    bhead = jnp.zeros((1, H), jnp.float32)
    bhead = bhead.at[0, :nat].set(params["atom_head"]["b"])
    bhead = bhead.at[0, nat:nat + 3].set(params["coord_head"]["b"])

    row, col = edge_index[0], edge_index[1]
    layers = params["layers"]
    h, ha, hb = _prologue_tc(onehot, table, tc_add, layers[0])

    nl = len(layers)
    for li in range(nl):
        lp = layers[li]
        need_coord = li < nl - 1
        ga, gb, rel = _gather(ha, hb, x, row, col)
        sm, sx = _edge_tc(ga, gb, rel, lp, need_coord)
        mi0, mi1, dx0, dx1 = _scatter(sm, sx, row, N, need_coord)
        if need_coord:
            h, x, ha, hb = _node_tc(h, mi0, mi1, dx0, dx1, x, lp,
                                    layers[li + 1])
        else:
            out = _final_tc(h, mi0, mi1, lp, params["norm"], whead, bhead)
    return out[:, :nat], out[:, nat:nat + 3]


# speedup vs baseline: 3.3724x; 3.3724x over previous
import functools
import math

import jax
import jax.numpy as jnp
from jax import lax
from jax.experimental import pallas as pl
from jax.experimental.pallas import tpu as pltpu
from jax.experimental.pallas import tpu_sc as plsc

H = 128
XP = 16
EB = 2000
NB = 1000


def _silu(v):
    return v * jax.nn.sigmoid(v)


def _edge_tc(ga, gb, rel, lp, need_coord):
    E = ga.shape[0]
    wd = lp["edge0"]["w"][2 * H].reshape(1, H)
    b0 = lp["edge0"]["b"].reshape(1, H)
    w1 = lp["edge1"]["w"]
    b1 = lp["edge1"]["b"].reshape(1, H)
    wa = jnp.zeros((H, 8), jnp.float32).at[:, 0].set(lp["att"]["w"][:, 0])
    ba = jnp.zeros((1, 8), jnp.float32).at[0, 0].set(lp["att"]["b"][0])
    wc0 = lp["coord0"]["w"]
    bc0 = lp["coord0"]["b"].reshape(1, H)
    wc1 = jnp.zeros((H, 8), jnp.float32).at[:, 0].set(lp["coord1"]["w"][:, 0])

    def body(ga_ref, gb_ref, rel_ref, wd_ref, b0_ref, w1_ref, b1_ref, wa_ref,
             ba_ref, wc0_ref, bc0_ref, wc1_ref, sm_ref, sx_ref):
        rel_v = rel_ref[...]
        dist = jnp.sqrt(jnp.sum(rel_v * rel_v, axis=1, keepdims=True))
        z = ga_ref[...] + gb_ref[...] + dist * wd_ref[...] + b0_ref[...]
        u = _silu(z)
        m = _silu(jnp.dot(u, w1_ref[...],
                          preferred_element_type=jnp.float32) + b1_ref[...])
        a8 = jnp.dot(m, wa_ref[...], preferred_element_type=jnp.float32)
        att = jax.nn.sigmoid(a8[:, :1] + ba_ref[...][:, :1])
        sm_ref[...] = att * m
        if need_coord:
            tcw = _silu(jnp.dot(m, wc0_ref[...],
                                preferred_element_type=jnp.float32)
                        + bc0_ref[...])
            c8 = jnp.dot(tcw, wc1_ref[...],
                         preferred_element_type=jnp.float32)
            sx_ref[...] = c8[:, :1] * (rel_v / (dist + 1e-8))
        else:
            sx_ref[...] = jnp.zeros_like(rel_v)

    grid = (E // EB,)
    row_spec = lambda w: pl.BlockSpec((EB, w), lambda i: (i, 0))
    full = lambda a: pl.BlockSpec(a.shape, lambda i: (0,) * a.ndim)
    sm, sx = pl.pallas_call(
        body,
        grid=grid,
        in_specs=[row_spec(H), row_spec(H), row_spec(XP), full(wd), full(b0),
                  full(w1), full(b1), full(wa), full(ba), full(wc0),
                  full(bc0), full(wc1)],
        out_specs=[row_spec(H), row_spec(XP)],
        out_shape=[jax.ShapeDtypeStruct((E, H), jnp.float32),
                   jax.ShapeDtypeStruct((E, XP), jnp.float32)],
    )(ga, gb, rel, wd, b0, w1, b1, wa, ba, wc0, bc0, wc1)
    return sm, sx


def _node_tc(h, mi0, mi1, dx0, dx1, x, lp, nxt):
    N = h.shape[0]
    wn0 = lp["node0"]["w"]
    wn0a, wn0b = wn0[:H], wn0[H:]
    bn0 = lp["node0"]["b"].reshape(1, H)
    wn1 = lp["node1"]["w"]
    bn1 = lp["node1"]["b"].reshape(1, H)
    w0 = nxt["edge0"]["w"]
    w0a, w0b = w0[:H], w0[H:2 * H]

    def body(h_ref, mi0_ref, mi1_ref, dx0_ref, dx1_ref, x_ref, wn0a_ref,
             wn0b_ref, bn0_ref, wn1_ref, bn1_ref, w0a_ref, w0b_ref,
             h_out, x_out, ha_out, hb_out):
        h_v = h_ref[...]
        mi = mi0_ref[...] + mi1_ref[...]
        u = _silu(jnp.dot(h_v, wn0a_ref[...],
                          preferred_element_type=jnp.float32)
                  + jnp.dot(mi, wn0b_ref[...],
                            preferred_element_type=jnp.float32)
                  + bn0_ref[...])
        hn = jnp.dot(u, wn1_ref[...],
                     preferred_element_type=jnp.float32) + bn1_ref[...]
        hv = h_v + hn
        h_out[...] = hv
        x_out[...] = x_ref[...] + dx0_ref[...] + dx1_ref[...]
        ha_out[...] = jnp.dot(hv, w0a_ref[...],
                              preferred_element_type=jnp.float32)
        hb_out[...] = jnp.dot(hv, w0b_ref[...],
                              preferred_element_type=jnp.float32)

    grid = (N // NB,)
    row_spec = lambda w: pl.BlockSpec((NB, w), lambda i: (i, 0))
    full = lambda a: pl.BlockSpec(a.shape, lambda i: (0,) * a.ndim)
    h2, x2, ha, hb = pl.pallas_call(
        body,
        grid=grid,
        in_specs=[row_spec(H), row_spec(H), row_spec(H), row_spec(XP),
                  row_spec(XP), row_spec(XP)] + [full(a) for a in
                  (wn0a, wn0b, bn0, wn1, bn1, w0a, w0b)],
        out_specs=[row_spec(H), row_spec(XP), row_spec(H), row_spec(H)],
        out_shape=[jax.ShapeDtypeStruct((N, H), jnp.float32),
                   jax.ShapeDtypeStruct((N, XP), jnp.float32),
                   jax.ShapeDtypeStruct((N, H), jnp.float32),
                   jax.ShapeDtypeStruct((N, H), jnp.float32)],
    )(h, mi0, mi1, dx0, dx1, x, wn0a, wn0b, bn0, wn1, bn1, w0a, w0b)
    return h2, x2, ha, hb


def _prologue_tc(onehot, table, tc_add, first):
    N = onehot.shape[0]
    w0 = first["edge0"]["w"]
    w0a, w0b = w0[:H], w0[H:2 * H]

    def body(oh_ref, tab_ref, add_ref, w0a_ref, w0b_ref,
             h_out, ha_out, hb_out):
        hv = jnp.dot(oh_ref[...], tab_ref[...],
                     preferred_element_type=jnp.float32) + add_ref[...]
        h_out[...] = hv
        ha_out[...] = jnp.dot(hv, w0a_ref[...],
                              preferred_element_type=jnp.float32)
        hb_out[...] = jnp.dot(hv, w0b_ref[...],
                              preferred_element_type=jnp.float32)

    grid = (N // NB,)
    row_spec = lambda w: pl.BlockSpec((NB, w), lambda i: (i, 0))
    full = lambda a: pl.BlockSpec(a.shape, lambda i: (0,) * a.ndim)
    h0, ha, hb = pl.pallas_call(
        body,
        grid=grid,
        in_specs=[row_spec(H), full(table), full(tc_add), full(w0a),
                  full(w0b)],
        out_specs=[row_spec(H), row_spec(H), row_spec(H)],
        out_shape=[jax.ShapeDtypeStruct((N, H), jnp.float32)] * 3,
    )(onehot, table, tc_add, w0a, w0b)
    return h0, ha, hb


def _final_tc(h, mi0, mi1, lp, norm, whead, bhead):
    N = h.shape[0]
    wn0 = lp["node0"]["w"]
    wn0a, wn0b = wn0[:H], wn0[H:]
    bn0 = lp["node0"]["b"].reshape(1, H)
    wn1 = lp["node1"]["w"]
    bn1 = lp["node1"]["b"].reshape(1, H)
    g = norm["g"].reshape(1, H)
    b = norm["b"].reshape(1, H)

    def body(h_ref, mi0_ref, mi1_ref, wn0a_ref, wn0b_ref, bn0_ref, wn1_ref,
             bn1_ref, g_ref, b_ref, wh_ref, bh_ref, out_ref):
        h_v = h_ref[...]
        mi = mi0_ref[...] + mi1_ref[...]
        u = _silu(jnp.dot(h_v, wn0a_ref[...],
                          preferred_element_type=jnp.float32)
                  + jnp.dot(mi, wn0b_ref[...],
                            preferred_element_type=jnp.float32)
                  + bn0_ref[...])
        hv = h_v + jnp.dot(u, wn1_ref[...],
                           preferred_element_type=jnp.float32) + bn1_ref[...]
        mu = jnp.mean(hv, axis=1, keepdims=True)
        var = jnp.mean((hv - mu) ** 2, axis=1, keepdims=True)
        hn = (hv - mu) / jnp.sqrt(var + 1e-5) * g_ref[...] + b_ref[...]
        out_ref[...] = jnp.dot(hn, wh_ref[...],
                               preferred_element_type=jnp.float32) + bh_ref[...]

    grid = (N // NB,)
    row_spec = lambda w: pl.BlockSpec((NB, w), lambda i: (i, 0))
    full = lambda a: pl.BlockSpec(a.shape, lambda i: (0,) * a.ndim)
    out = pl.pallas_call(
        body,
        grid=grid,
        in_specs=[row_spec(H), row_spec(H), row_spec(H)] + [full(a) for a in
                  (wn0a, wn0b, bn0, wn1, bn1, g, b, whead, bhead)],
        out_specs=row_spec(H),
        out_shape=jax.ShapeDtypeStruct((N, H), jnp.float32),
    )(h, mi0, mi1, wn0a, wn0b, bn0, wn1, bn1, g, b, whead, bhead)
    return out


NW = 32
SC_C = 128


def _gather_sc(ha, hb, x, row, col):
    N = ha.shape[0]
    E = row.shape[0]
    epw = E // NW
    nch = epw // SC_C
    tail = epw - nch * SC_C
    mesh = plsc.VectorSubcoreMesh(core_axis_name="c", subcore_axis_name="s")

    @functools.partial(
        pl.kernel, mesh=mesh,
        compiler_params=pltpu.CompilerParams(use_tc_tiling_on_sc=False),
        out_type=[jax.ShapeDtypeStruct((E, H), jnp.float32),
                  jax.ShapeDtypeStruct((E, H), jnp.float32),
                  jax.ShapeDtypeStruct((E, XP), jnp.float32)],
        scratch_types=[pltpu.VMEM((SC_C,), jnp.int32),
                       pltpu.VMEM((SC_C,), jnp.int32),
                       pltpu.VMEM((tail,), jnp.int32),
                       pltpu.VMEM((tail,), jnp.int32),
                       pltpu.VMEM((SC_C, H), jnp.float32),
                       pltpu.VMEM((SC_C, H), jnp.float32),
                       pltpu.VMEM((SC_C, XP), jnp.float32),
                       pltpu.VMEM((SC_C, XP), jnp.float32),
                       pltpu.SemaphoreType.DMA],
    )
    def k(ha_hbm, hb_hbm, x_hbm, row_hbm, col_hbm, ga_hbm, gb_hbm, rel_hbm,
          idxr, idxc, idxrt, idxct, bufa, bufb, bxr, bxc, sem):
        c = lax.axis_index("c")
        s = lax.axis_index("s")
        wbase = (s * 2 + c) * epw

        def chunk(base, cc, ir, ic):
            pltpu.sync_copy(row_hbm.at[pl.ds(base, cc)], ir)
            pltpu.sync_copy(col_hbm.at[pl.ds(base, cc)], ic)
            ca = pltpu.async_copy(ha_hbm.at[ir], bufa.at[pl.ds(0, cc)], sem)
            cb = pltpu.async_copy(hb_hbm.at[ic], bufb.at[pl.ds(0, cc)], sem)
            cxr = pltpu.async_copy(x_hbm.at[ir], bxr.at[pl.ds(0, cc)], sem)
            cxc = pltpu.async_copy(x_hbm.at[ic], bxc.at[pl.ds(0, cc)], sem)
            ca.wait(); cb.wait(); cxr.wait(); cxc.wait()

            def sub(j, _):
                bxr[j] = bxr[j] - bxc[j]
                return 0
            lax.fori_loop(0, cc, sub, 0, unroll=4)
            pltpu.sync_copy(bufa.at[pl.ds(0, cc)], ga_hbm.at[pl.ds(base, cc)])
            pltpu.sync_copy(bufb.at[pl.ds(0, cc)], gb_hbm.at[pl.ds(base, cc)])
            pltpu.sync_copy(bxr.at[pl.ds(0, cc)], rel_hbm.at[pl.ds(base, cc)])

        def loop_body(kk, _):
            chunk(wbase + kk * SC_C, SC_C, idxr, idxc)
            return 0
        lax.fori_loop(0, nch, loop_body, 0)
        if tail:
            chunk(wbase + nch * SC_C, tail, idxrt, idxct)

    return k(ha, hb, x, row, col)


def _scatter_sc(sm, sx, row, zeros_h, zeros_x, need_coord):
    E = sm.shape[0]
    N = zeros_h.shape[0]
    epw = E // NW
    nch = epw // SC_C
    tail = epw - nch * SC_C
    nps = N // 16
    mesh = plsc.VectorSubcoreMesh(core_axis_name="c", subcore_axis_name="s")

    out_type = [jax.ShapeDtypeStruct((2, N, H), jnp.float32)]
    scratch = [pltpu.VMEM((SC_C,), jnp.int32),
               pltpu.VMEM((tail,), jnp.int32),
               pltpu.VMEM((SC_C, H), jnp.float32),
               pltpu.VMEM((SC_C, XP), jnp.float32),
               pltpu.VMEM_SHARED((N, H), jnp.float32),
               pltpu.VMEM_SHARED((N, XP), jnp.float32),
               pltpu.SemaphoreType.DMA]
    if need_coord:
        out_type.append(jax.ShapeDtypeStruct((2, N, XP), jnp.float32))

    @functools.partial(pl.kernel, mesh=mesh, out_type=out_type,
                       compiler_params=pltpu.CompilerParams(
                           use_tc_tiling_on_sc=False),
                       scratch_types=scratch)
    def k(sm_hbm, sx_hbm, row_hbm, zh_hbm, zx_hbm, *rest):
        if need_coord:
            mi_hbm, dx_hbm = rest[0], rest[1]
            rest = rest[2:]
        else:
            mi_hbm, dx_hbm = rest[0], None
            rest = rest[1:]
        idx, idxt, bufm, bufx, acch, accx, sem = rest
        c = lax.axis_index("c")
        s = lax.axis_index("s")
        stripe = pl.ds(s * nps, nps)
        pltpu.sync_copy(zh_hbm.at[stripe], acch.at[stripe])
        if need_coord:
            pltpu.sync_copy(zx_hbm.at[stripe], accx.at[stripe])
        plsc.subcore_barrier()
        wbase = (s * 2 + c) * epw

        def chunk(base, cc, ir):
            pltpu.sync_copy(row_hbm.at[pl.ds(base, cc)], ir)
            cm = pltpu.async_copy(sm_hbm.at[pl.ds(base, cc)],
                                  bufm.at[pl.ds(0, cc)], sem)
            if need_coord:
                cx = pltpu.async_copy(sx_hbm.at[pl.ds(base, cc)],
                                      bufx.at[pl.ds(0, cc)], sem)
            cm.wait()
            pltpu.sync_copy(bufm.at[pl.ds(0, cc)], acch.at[ir], add=True)
            if need_coord:
                cx.wait()
                pltpu.sync_copy(bufx.at[pl.ds(0, cc)], accx.at[ir], add=True)

        def loop_body(kk, _):
            chunk(wbase + kk * SC_C, SC_C, idx)
            return 0
        lax.fori_loop(0, nch, loop_body, 0)
        if tail:
            chunk(wbase + nch * SC_C, tail, idxt)
        plsc.subcore_barrier()
        pltpu.sync_copy(acch.at[stripe], mi_hbm.at[c, stripe])
        if need_coord:
            pltpu.sync_copy(accx.at[stripe], dx_hbm.at[c, stripe])

    outs = k(sm, sx, row, zeros_h, zeros_x)
    if need_coord:
        mi2, dx2 = outs
        return mi2[0], mi2[1], dx2[0], dx2[1]
    mi2 = outs if isinstance(outs, jax.Array) else outs[0]
    return mi2[0], mi2[1], zeros_x, zeros_x


def _apply(p, v):
    y = v @ p["w"]
    if "b" in p:
        y = y + p["b"]
    return y


def kernel(atom_types, coords, edge_index, t, anchor_features,
           distance_constraints, coordination_constraints, params):
    N = coords.shape[0]

    half = H // 2
    freqs = jnp.exp(jnp.arange(half, dtype=jnp.float32)
                    * (-(math.log(10000.0) / (half - 1))))
    te = t.astype(jnp.float32)[:, None] * freqs[None, :]
    te = jnp.concatenate([jnp.sin(te), jnp.cos(te)], axis=-1)
    te = _apply(params["time_mlp"][1],
                jax.nn.silu(_apply(params["time_mlp"][0], te)))
    a = _apply(params["cond_anchor"][1], jax.nn.silu(
        _apply(params["cond_anchor"][0], anchor_features))).mean(
            axis=0, keepdims=True)
    d = _apply(params["cond_dist"][1], jax.nn.silu(
        _apply(params["cond_dist"][0], distance_constraints))).mean(
            axis=0, keepdims=True)
    c = _apply(params["cond_coord"][1], jax.nn.silu(
        _apply(params["cond_coord"][0], coordination_constraints))).mean(
            axis=0, keepdims=True)
    comb = jnp.concatenate([a, d, c], axis=-1)
    fz = _apply(params["fusion_lin0"], comb)
    mu = fz.mean(axis=-1, keepdims=True)
    var = ((fz - mu) ** 2).mean(axis=-1, keepdims=True)
    fz = (fz - mu) / jnp.sqrt(var + 1e-5) * params["fusion_ln"]["g"] \
        + params["fusion_ln"]["b"]
    ce = _apply(params["fusion_lin1"], jax.nn.silu(fz))
    tc_add = _apply(params["time_proj"], te) + _apply(params["cond_proj"], ce)

    x = jnp.zeros((N, XP), jnp.float32).at[:, :3].set(coords)
    onehot = jax.nn.one_hot(atom_types, H, dtype=jnp.float32)
    table = jnp.zeros((H, H), jnp.float32).at[
        :params["atom_embed"].shape[0]].set(params["atom_embed"])
    nat = params["atom_head"]["w"].shape[1]
    whead = jnp.zeros((H, H), jnp.float32)
    whead = whead.at[:, :nat].set(params["atom_head"]["w"])
    whead = whead.at[:, nat:nat + 3].set(params["coord_head"]["w"])
    bhead = jnp.zeros((1, H), jnp.float32)
    bhead = bhead.at[0, :nat].set(params["atom_head"]["b"])
    bhead = bhead.at[0, nat:nat + 3].set(params["coord_head"]["b"])

    row, col = edge_index[0], edge_index[1]
    layers = params["layers"]
    h, ha, hb = _prologue_tc(onehot, table, tc_add, layers[0])
    zeros_h = jnp.zeros((N, H), jnp.float32)
    zeros_x = jnp.zeros((N, XP), jnp.float32)

    nl = len(layers)
    for li in range(nl):
        lp = layers[li]
        need_coord = li < nl - 1
        ga, gb, rel = _gather_sc(ha, hb, x, row, col)
        sm, sx = _edge_tc(ga, gb, rel, lp, need_coord)
        mi0, mi1, dx0, dx1 = _scatter_sc(sm, sx, row, zeros_h, zeros_x,
                                         need_coord)
        if need_coord:
            h, x, ha, hb = _node_tc(h, mi0, mi1, dx0, dx1, x, lp,
                                    layers[li + 1])
        else:
            out = _final_tc(h, mi0, mi1, lp, params["norm"], whead, bhead)
    return out[:, :nat], out[:, nat:nat + 3]


# speedup vs baseline: 4.3213x; 1.2814x over previous
import functools
import math

import jax
import jax.numpy as jnp
from jax import lax
from jax.experimental import pallas as pl
from jax.experimental.pallas import tpu as pltpu
from jax.experimental.pallas import tpu_sc as plsc

H = 128
XP = 16
EB = 2000
NB = 1000


def _silu(v):
    return v * jax.nn.sigmoid(v)


def _edge_tc(ga, gb, rel, lp, need_coord):
    E = ga.shape[0]
    wd = lp["edge0"]["w"][2 * H].reshape(1, H)
    b0 = lp["edge0"]["b"].reshape(1, H)
    w1 = lp["edge1"]["w"]
    b1 = lp["edge1"]["b"].reshape(1, H)
    wa = jnp.zeros((H, 8), jnp.float32).at[:, 0].set(lp["att"]["w"][:, 0])
    ba = jnp.zeros((1, 8), jnp.float32).at[0, 0].set(lp["att"]["b"][0])
    wc0 = lp["coord0"]["w"]
    bc0 = lp["coord0"]["b"].reshape(1, H)
    wc1 = jnp.zeros((H, 8), jnp.float32).at[:, 0].set(lp["coord1"]["w"][:, 0])

    def body(ga_ref, gb_ref, rel_ref, wd_ref, b0_ref, w1_ref, b1_ref, wa_ref,
             ba_ref, wc0_ref, bc0_ref, wc1_ref, sm_ref, sx_ref):
        rel_v = rel_ref[...]
        dist = jnp.sqrt(jnp.sum(rel_v * rel_v, axis=1, keepdims=True))
        z = ga_ref[...] + gb_ref[...] + dist * wd_ref[...] + b0_ref[...]
        u = _silu(z)
        m = _silu(jnp.dot(u, w1_ref[...],
                          preferred_element_type=jnp.float32) + b1_ref[...])
        a8 = jnp.dot(m, wa_ref[...], preferred_element_type=jnp.float32)
        att = jax.nn.sigmoid(a8[:, :1] + ba_ref[...][:, :1])
        sm_ref[...] = att * m
        if need_coord:
            tcw = _silu(jnp.dot(m, wc0_ref[...],
                                preferred_element_type=jnp.float32)
                        + bc0_ref[...])
            c8 = jnp.dot(tcw, wc1_ref[...],
                         preferred_element_type=jnp.float32)
            sx_ref[...] = c8[:, :1] * (rel_v / (dist + 1e-8))
        else:
            sx_ref[...] = jnp.zeros_like(rel_v)

    grid = (E // EB,)
    row_spec = lambda w: pl.BlockSpec((EB, w), lambda i: (i, 0))
    full = lambda a: pl.BlockSpec(a.shape, lambda i: (0,) * a.ndim)
    sm, sx = pl.pallas_call(
        body,
        grid=grid,
        in_specs=[row_spec(H), row_spec(H), row_spec(XP), full(wd), full(b0),
                  full(w1), full(b1), full(wa), full(ba), full(wc0),
                  full(bc0), full(wc1)],
        out_specs=[row_spec(H), row_spec(XP)],
        out_shape=[jax.ShapeDtypeStruct((E, H), jnp.float32),
                   jax.ShapeDtypeStruct((E, XP), jnp.float32)],
    )(ga, gb, rel, wd, b0, w1, b1, wa, ba, wc0, bc0, wc1)
    return sm, sx


def _node_tc(h, mi0, mi1, dx0, dx1, x, lp, nxt):
    N = h.shape[0]
    wn0 = lp["node0"]["w"]
    wn0a, wn0b = wn0[:H], wn0[H:]
    bn0 = lp["node0"]["b"].reshape(1, H)
    wn1 = lp["node1"]["w"]
    bn1 = lp["node1"]["b"].reshape(1, H)
    w0 = nxt["edge0"]["w"]
    w0a, w0b = w0[:H], w0[H:2 * H]

    def body(h_ref, mi0_ref, mi1_ref, dx0_ref, dx1_ref, x_ref, wn0a_ref,
             wn0b_ref, bn0_ref, wn1_ref, bn1_ref, w0a_ref, w0b_ref,
             h_out, x_out, ha_out, hb_out):
        h_v = h_ref[...]
        mi = mi0_ref[...] + mi1_ref[...]
        u = _silu(jnp.dot(h_v, wn0a_ref[...],
                          preferred_element_type=jnp.float32)
                  + jnp.dot(mi, wn0b_ref[...],
                            preferred_element_type=jnp.float32)
                  + bn0_ref[...])
        hn = jnp.dot(u, wn1_ref[...],
                     preferred_element_type=jnp.float32) + bn1_ref[...]
        hv = h_v + hn
        h_out[...] = hv
        x_out[...] = x_ref[...] + dx0_ref[...] + dx1_ref[...]
        ha_out[...] = jnp.dot(hv, w0a_ref[...],
                              preferred_element_type=jnp.float32)
        hb_out[...] = jnp.dot(hv, w0b_ref[...],
                              preferred_element_type=jnp.float32)

    grid = (N // NB,)
    row_spec = lambda w: pl.BlockSpec((NB, w), lambda i: (i, 0))
    full = lambda a: pl.BlockSpec(a.shape, lambda i: (0,) * a.ndim)
    h2, x2, ha, hb = pl.pallas_call(
        body,
        grid=grid,
        in_specs=[row_spec(H), row_spec(H), row_spec(H), row_spec(XP),
                  row_spec(XP), row_spec(XP)] + [full(a) for a in
                  (wn0a, wn0b, bn0, wn1, bn1, w0a, w0b)],
        out_specs=[row_spec(H), row_spec(XP), row_spec(H), row_spec(H)],
        out_shape=[jax.ShapeDtypeStruct((N, H), jnp.float32),
                   jax.ShapeDtypeStruct((N, XP), jnp.float32),
                   jax.ShapeDtypeStruct((N, H), jnp.float32),
                   jax.ShapeDtypeStruct((N, H), jnp.float32)],
    )(h, mi0, mi1, dx0, dx1, x, wn0a, wn0b, bn0, wn1, bn1, w0a, w0b)
    return h2, x2, ha, hb


def _prologue_tc(onehot, table, tc_add, first):
    N = onehot.shape[0]
    w0 = first["edge0"]["w"]
    w0a, w0b = w0[:H], w0[H:2 * H]

    def body(oh_ref, tab_ref, add_ref, w0a_ref, w0b_ref,
             h_out, ha_out, hb_out):
        hv = jnp.dot(oh_ref[...], tab_ref[...],
                     preferred_element_type=jnp.float32) + add_ref[...]
        h_out[...] = hv
        ha_out[...] = jnp.dot(hv, w0a_ref[...],
                              preferred_element_type=jnp.float32)
        hb_out[...] = jnp.dot(hv, w0b_ref[...],
                              preferred_element_type=jnp.float32)

    grid = (N // NB,)
    row_spec = lambda w: pl.BlockSpec((NB, w), lambda i: (i, 0))
    full = lambda a: pl.BlockSpec(a.shape, lambda i: (0,) * a.ndim)
    h0, ha, hb = pl.pallas_call(
        body,
        grid=grid,
        in_specs=[row_spec(H), full(table), full(tc_add), full(w0a),
                  full(w0b)],
        out_specs=[row_spec(H), row_spec(H), row_spec(H)],
        out_shape=[jax.ShapeDtypeStruct((N, H), jnp.float32)] * 3,
    )(onehot, table, tc_add, w0a, w0b)
    return h0, ha, hb


def _final_tc(h, mi0, mi1, lp, norm, whead, bhead):
    N = h.shape[0]
    wn0 = lp["node0"]["w"]
    wn0a, wn0b = wn0[:H], wn0[H:]
    bn0 = lp["node0"]["b"].reshape(1, H)
    wn1 = lp["node1"]["w"]
    bn1 = lp["node1"]["b"].reshape(1, H)
    g = norm["g"].reshape(1, H)
    b = norm["b"].reshape(1, H)

    def body(h_ref, mi0_ref, mi1_ref, wn0a_ref, wn0b_ref, bn0_ref, wn1_ref,
             bn1_ref, g_ref, b_ref, wh_ref, bh_ref, out_ref):
        h_v = h_ref[...]
        mi = mi0_ref[...] + mi1_ref[...]
        u = _silu(jnp.dot(h_v, wn0a_ref[...],
                          preferred_element_type=jnp.float32)
                  + jnp.dot(mi, wn0b_ref[...],
                            preferred_element_type=jnp.float32)
                  + bn0_ref[...])
        hv = h_v + jnp.dot(u, wn1_ref[...],
                           preferred_element_type=jnp.float32) + bn1_ref[...]
        mu = jnp.mean(hv, axis=1, keepdims=True)
        var = jnp.mean((hv - mu) ** 2, axis=1, keepdims=True)
        hn = (hv - mu) / jnp.sqrt(var + 1e-5) * g_ref[...] + b_ref[...]
        out_ref[...] = jnp.dot(hn, wh_ref[...],
                               preferred_element_type=jnp.float32) + bh_ref[...]

    grid = (N // NB,)
    row_spec = lambda w: pl.BlockSpec((NB, w), lambda i: (i, 0))
    full = lambda a: pl.BlockSpec(a.shape, lambda i: (0,) * a.ndim)
    out = pl.pallas_call(
        body,
        grid=grid,
        in_specs=[row_spec(H), row_spec(H), row_spec(H)] + [full(a) for a in
                  (wn0a, wn0b, bn0, wn1, bn1, g, b, whead, bhead)],
        out_specs=row_spec(H),
        out_shape=jax.ShapeDtypeStruct((N, H), jnp.float32),
    )(h, mi0, mi1, wn0a, wn0b, bn0, wn1, bn1, g, b, whead, bhead)
    return out


NW = 32
SC_C = 128


def _gather_sc(ha, hb, x, row, col):
    N = ha.shape[0]
    E = row.shape[0]
    epw = E // NW
    nch = epw // SC_C
    tail = epw - nch * SC_C
    mesh = plsc.VectorSubcoreMesh(core_axis_name="c", subcore_axis_name="s")

    @functools.partial(
        pl.kernel, mesh=mesh,
        compiler_params=pltpu.CompilerParams(use_tc_tiling_on_sc=False),
        out_type=[jax.ShapeDtypeStruct((E, H), jnp.float32),
                  jax.ShapeDtypeStruct((E, H), jnp.float32),
                  jax.ShapeDtypeStruct((E, XP), jnp.float32)],
        scratch_types=[pltpu.VMEM((2, SC_C), jnp.int32),
                       pltpu.VMEM((2, SC_C), jnp.int32),
                       pltpu.VMEM((tail,), jnp.int32),
                       pltpu.VMEM((tail,), jnp.int32),
                       pltpu.VMEM((2, SC_C, H), jnp.float32),
                       pltpu.VMEM((2, SC_C, H), jnp.float32),
                       pltpu.VMEM((2, SC_C, XP), jnp.float32),
                       pltpu.VMEM((2, SC_C, XP), jnp.float32),
                       pltpu.SemaphoreType.DMA((2,)),
                       pltpu.SemaphoreType.DMA((2,)),
                       pltpu.SemaphoreType.DMA((2,)),
                       pltpu.SemaphoreType.DMA],
    )
    def k(ha_hbm, hb_hbm, x_hbm, row_hbm, col_hbm, ga_hbm, gb_hbm, rel_hbm,
          idxr, idxc, idxrt, idxct, bufa, bufb, bxr, bxc, isem, gsem, ssem,
          tsem):
        c = lax.axis_index("c")
        s = lax.axis_index("s")
        wbase = (s * 2 + c) * epw

        def start_idx(kk, b):
            base = wbase + kk * SC_C
            pltpu.async_copy(row_hbm.at[pl.ds(base, SC_C)], idxr.at[b],
                             isem.at[b])
            pltpu.async_copy(col_hbm.at[pl.ds(base, SC_C)], idxc.at[b],
                             isem.at[b])

        def start_gather(b):
            pltpu.make_async_copy(row_hbm.at[pl.ds(0, SC_C)], idxr.at[b],
                                  isem.at[b]).wait()
            pltpu.make_async_copy(row_hbm.at[pl.ds(0, SC_C)], idxc.at[b],
                                  isem.at[b]).wait()
            pltpu.async_copy(ha_hbm.at[idxr.at[b]], bufa.at[b], gsem.at[b])
            pltpu.async_copy(hb_hbm.at[idxc.at[b]], bufb.at[b], gsem.at[b])
            pltpu.async_copy(x_hbm.at[idxr.at[b]], bxr.at[b], gsem.at[b])
            pltpu.async_copy(x_hbm.at[idxc.at[b]], bxc.at[b], gsem.at[b])

        def wait_gather(b):
            pltpu.make_async_copy(ha_hbm.at[pl.ds(0, SC_C)], bufa.at[b],
                                  gsem.at[b]).wait()
            pltpu.make_async_copy(hb_hbm.at[pl.ds(0, SC_C)], bufb.at[b],
                                  gsem.at[b]).wait()
            pltpu.make_async_copy(x_hbm.at[pl.ds(0, SC_C)], bxr.at[b],
                                  gsem.at[b]).wait()
            pltpu.make_async_copy(x_hbm.at[pl.ds(0, SC_C)], bxc.at[b],
                                  gsem.at[b]).wait()

        def subloop(b, cc):
            def sub(j, _):
                bxr[b, j] = bxr[b, j] - bxc[b, j]
                return 0
            lax.fori_loop(0, cc, sub, 0, unroll=4)

        def start_store(kk, b):
            base = wbase + kk * SC_C
            pltpu.async_copy(bufa.at[b], ga_hbm.at[pl.ds(base, SC_C)],
                             ssem.at[b])
            pltpu.async_copy(bufb.at[b], gb_hbm.at[pl.ds(base, SC_C)],
                             ssem.at[b])
            pltpu.async_copy(bxr.at[b], rel_hbm.at[pl.ds(base, SC_C)],
                             ssem.at[b])

        def wait_store(b):
            pltpu.make_async_copy(bufa.at[b], ga_hbm.at[pl.ds(0, SC_C)],
                                  ssem.at[b]).wait()
            pltpu.make_async_copy(bufb.at[b], gb_hbm.at[pl.ds(0, SC_C)],
                                  ssem.at[b]).wait()
            pltpu.make_async_copy(bxr.at[b], rel_hbm.at[pl.ds(0, SC_C)],
                                  ssem.at[b]).wait()

        start_idx(0, 0)
        start_gather(0)
        start_idx(1, 1)

        def body(kk, _):
            b = lax.rem(kk, 2)
            nb = 1 - b

            @pl.when(kk > 0)
            def _():
                wait_store(nb)

            @pl.when(kk + 1 < nch)
            def _():
                start_gather(nb)
            wait_gather(b)

            @pl.when(kk + 2 < nch)
            def _():
                start_idx(kk + 2, b)
            subloop(b, SC_C)
            start_store(kk, b)
            return 0

        lax.fori_loop(0, nch, body, 0)
        wait_store(lax.rem(nch - 1, 2))
        if tail:
            base = wbase + nch * SC_C
            pltpu.sync_copy(row_hbm.at[pl.ds(base, tail)], idxrt)
            pltpu.sync_copy(col_hbm.at[pl.ds(base, tail)], idxct)
            ts = pl.ds(0, tail)
            ca = pltpu.async_copy(ha_hbm.at[idxrt], bufa.at[0, ts], tsem)
            cb = pltpu.async_copy(hb_hbm.at[idxct], bufb.at[0, ts], tsem)
            cr = pltpu.async_copy(x_hbm.at[idxrt], bxr.at[0, ts], tsem)
            cc2 = pltpu.async_copy(x_hbm.at[idxct], bxc.at[0, ts], tsem)
            ca.wait(); cb.wait(); cr.wait(); cc2.wait()
            subloop(0, tail)
            pltpu.sync_copy(bufa.at[0, ts], ga_hbm.at[pl.ds(base, tail)])
            pltpu.sync_copy(bufb.at[0, ts], gb_hbm.at[pl.ds(base, tail)])
            pltpu.sync_copy(bxr.at[0, ts], rel_hbm.at[pl.ds(base, tail)])

    return k(ha, hb, x, row, col)


def _scatter_sc(sm, sx, row, zeros_h, zeros_x, need_coord):
    E = sm.shape[0]
    N = zeros_h.shape[0]
    epw = E // NW
    nch = epw // SC_C
    tail = epw - nch * SC_C
    nps = N // 16
    mesh = plsc.VectorSubcoreMesh(core_axis_name="c", subcore_axis_name="s")

    out_type = [jax.ShapeDtypeStruct((2, N, H), jnp.float32)]
    scratch = [pltpu.VMEM((2, SC_C), jnp.int32),
               pltpu.VMEM((tail,), jnp.int32),
               pltpu.VMEM((2, SC_C, H), jnp.float32),
               pltpu.VMEM((2, SC_C, XP), jnp.float32),
               pltpu.VMEM_SHARED((N, H), jnp.float32),
               pltpu.VMEM_SHARED((N, XP), jnp.float32),
               pltpu.SemaphoreType.DMA((2,)),
               pltpu.SemaphoreType.DMA((2,)),
               pltpu.SemaphoreType.DMA]
    if need_coord:
        out_type.append(jax.ShapeDtypeStruct((2, N, XP), jnp.float32))

    @functools.partial(pl.kernel, mesh=mesh, out_type=out_type,
                       compiler_params=pltpu.CompilerParams(
                           use_tc_tiling_on_sc=False),
                       scratch_types=scratch)
    def k(sm_hbm, sx_hbm, row_hbm, zh_hbm, zx_hbm, *rest):
        if need_coord:
            mi_hbm, dx_hbm = rest[0], rest[1]
            rest = rest[2:]
        else:
            mi_hbm, dx_hbm = rest[0], None
            rest = rest[1:]
        idx, idxt, bufm, bufx, acch, accx, lsem, asem, tsem = rest
        c = lax.axis_index("c")
        s = lax.axis_index("s")
        stripe = pl.ds(s * nps, nps)
        pltpu.sync_copy(zh_hbm.at[stripe], acch.at[stripe])
        if need_coord:
            pltpu.sync_copy(zx_hbm.at[stripe], accx.at[stripe])
        plsc.subcore_barrier()
        wbase = (s * 2 + c) * epw

        def start_loads(kk, b):
            base = wbase + kk * SC_C
            pltpu.async_copy(row_hbm.at[pl.ds(base, SC_C)], idx.at[b],
                             lsem.at[b])
            pltpu.async_copy(sm_hbm.at[pl.ds(base, SC_C)], bufm.at[b],
                             lsem.at[b])
            if need_coord:
                pltpu.async_copy(sx_hbm.at[pl.ds(base, SC_C)], bufx.at[b],
                                 lsem.at[b])

        def wait_loads(b):
            pltpu.make_async_copy(row_hbm.at[pl.ds(0, SC_C)], idx.at[b],
                                  lsem.at[b]).wait()
            pltpu.make_async_copy(sm_hbm.at[pl.ds(0, SC_C)], bufm.at[b],
                                  lsem.at[b]).wait()
            if need_coord:
                pltpu.make_async_copy(sx_hbm.at[pl.ds(0, SC_C)], bufx.at[b],
                                      lsem.at[b]).wait()

        def start_adds(b):
            pltpu.async_copy(bufm.at[b], acch.at[idx.at[b]], asem.at[b],
                             add=True)
            if need_coord:
                pltpu.async_copy(bufx.at[b], accx.at[idx.at[b]], asem.at[b],
                                 add=True)

        def wait_adds(b):
            pltpu.make_async_copy(sm_hbm.at[pl.ds(0, SC_C)], bufm.at[b],
                                  asem.at[b]).wait()
            if need_coord:
                pltpu.make_async_copy(sx_hbm.at[pl.ds(0, SC_C)], bufx.at[b],
                                      asem.at[b]).wait()

        start_loads(0, 0)

        def body(kk, _):
            b = lax.rem(kk, 2)
            nb = 1 - b

            @pl.when(kk > 0)
            def _():
                wait_adds(nb)

            @pl.when(kk + 1 < nch)
            def _():
                start_loads(kk + 1, nb)
            wait_loads(b)
            start_adds(b)
            return 0

        lax.fori_loop(0, nch, body, 0)
        wait_adds(lax.rem(nch - 1, 2))
        if tail:
            base = wbase + nch * SC_C
            ts = pl.ds(0, tail)
            pltpu.sync_copy(row_hbm.at[pl.ds(base, tail)], idxt)
            cm = pltpu.async_copy(sm_hbm.at[pl.ds(base, tail)],
                                  bufm.at[0, ts], tsem)
            if need_coord:
                cx = pltpu.async_copy(sx_hbm.at[pl.ds(base, tail)],
                                      bufx.at[0, ts], tsem)
            cm.wait()
            pltpu.sync_copy(bufm.at[0, ts], acch.at[idxt], add=True)
            if need_coord:
                cx.wait()
                pltpu.sync_copy(bufx.at[0, ts], accx.at[idxt], add=True)
        plsc.subcore_barrier()
        pltpu.sync_copy(acch.at[stripe], mi_hbm.at[c, stripe])
        if need_coord:
            pltpu.sync_copy(accx.at[stripe], dx_hbm.at[c, stripe])

    outs = k(sm, sx, row, zeros_h, zeros_x)
    if need_coord:
        mi2, dx2 = outs
        return mi2[0], mi2[1], dx2[0], dx2[1]
    mi2 = outs if isinstance(outs, jax.Array) else outs[0]
    return mi2[0], mi2[1], zeros_x, zeros_x


def _apply(p, v):
    y = v @ p["w"]
    if "b" in p:
        y = y + p["b"]
    return y


def kernel(atom_types, coords, edge_index, t, anchor_features,
           distance_constraints, coordination_constraints, params):
    N = coords.shape[0]

    half = H // 2
    freqs = jnp.exp(jnp.arange(half, dtype=jnp.float32)
                    * (-(math.log(10000.0) / (half - 1))))
    te = t.astype(jnp.float32)[:, None] * freqs[None, :]
    te = jnp.concatenate([jnp.sin(te), jnp.cos(te)], axis=-1)
    te = _apply(params["time_mlp"][1],
                jax.nn.silu(_apply(params["time_mlp"][0], te)))
    a = _apply(params["cond_anchor"][1], jax.nn.silu(
        _apply(params["cond_anchor"][0], anchor_features))).mean(
            axis=0, keepdims=True)
    d = _apply(params["cond_dist"][1], jax.nn.silu(
        _apply(params["cond_dist"][0], distance_constraints))).mean(
            axis=0, keepdims=True)
    c = _apply(params["cond_coord"][1], jax.nn.silu(
        _apply(params["cond_coord"][0], coordination_constraints))).mean(
            axis=0, keepdims=True)
    comb = jnp.concatenate([a, d, c], axis=-1)
    fz = _apply(params["fusion_lin0"], comb)
    mu = fz.mean(axis=-1, keepdims=True)
    var = ((fz - mu) ** 2).mean(axis=-1, keepdims=True)
    fz = (fz - mu) / jnp.sqrt(var + 1e-5) * params["fusion_ln"]["g"] \
        + params["fusion_ln"]["b"]
    ce = _apply(params["fusion_lin1"], jax.nn.silu(fz))
    tc_add = _apply(params["time_proj"], te) + _apply(params["cond_proj"], ce)

    x = jnp.zeros((N, XP), jnp.float32).at[:, :3].set(coords)
    onehot = jax.nn.one_hot(atom_types, H, dtype=jnp.float32)
    table = jnp.zeros((H, H), jnp.float32).at[
        :params["atom_embed"].shape[0]].set(params["atom_embed"])
    nat = params["atom_head"]["w"].shape[1]
    whead = jnp.zeros((H, H), jnp.float32)
    whead = whead.at[:, :nat].set(params["atom_head"]["w"])
    whead = whead.at[:, nat:nat + 3].set(params["coord_head"]["w"])
    bhead = jnp.zeros((1, H), jnp.float32)
    bhead = bhead.at[0, :nat].set(params["atom_head"]["b"])
    bhead = bhead.at[0, nat:nat + 3].set(params["coord_head"]["b"])

    row, col = edge_index[0], edge_index[1]
    layers = params["layers"]
    h, ha, hb = _prologue_tc(onehot, table, tc_add, layers[0])
    zeros_h = jnp.zeros((N, H), jnp.float32)
    zeros_x = jnp.zeros((N, XP), jnp.float32)

    nl = len(layers)
    for li in range(nl):
        lp = layers[li]
        need_coord = li < nl - 1
        ga, gb, rel = _gather_sc(ha, hb, x, row, col)
        sm, sx = _edge_tc(ga, gb, rel, lp, need_coord)
        mi0, mi1, dx0, dx1 = _scatter_sc(sm, sx, row, zeros_h, zeros_x,
                                         need_coord)
        if need_coord:
            h, x, ha, hb = _node_tc(h, mi0, mi1, dx0, dx1, x, lp,
                                    layers[li + 1])
        else:
            out = _final_tc(h, mi0, mi1, lp, params["norm"], whead, bhead)
    return out[:, :nat], out[:, nat:nat + 3]
